# baseline (speedup 1.0000x reference)
import functools

import jax
import jax.numpy as jnp
from jax import lax
from jax.experimental import pallas as pl
from jax.experimental.pallas import tpu as pltpu
from jax.experimental.pallas import tpu_sc as plsc

_EMB = (64, 32, 32, 16)
_OFF = (0, 64, 96, 128)
_PADW = 128
_DTOT = 144
_B, _L = 4096, 50
_N = _B * _L
_NW = 32
_ROWS_PER_W = _N // _NW
_CHUNK = 80
_NCHUNK = _ROWS_PER_W // _CHUNK
_DEPTH = 2
_NOUTER = _NCHUNK // _DEPTH

_mesh = plsc.VectorSubcoreMesh(core_axis_name="c", subcore_axis_name="s")


@functools.partial(
    pl.kernel,
    mesh=_mesh,
    out_type=jax.ShapeDtypeStruct((_N, _DTOT), jnp.float32),
    scratch_types=(
        [pltpu.VMEM((_ROWS_PER_W,), jnp.int32) for _ in range(4)]
        + [[pltpu.VMEM((_CHUNK, _PADW), jnp.float32) for _ in range(4)]
           for _ in range(_DEPTH)]
        + [[pltpu.SemaphoreType.DMA for _ in range(_DEPTH)],
           pltpu.SemaphoreType.DMA]
    ),
    compiler_params=pltpu.CompilerParams(use_tc_tiling_on_sc=False),
)
def _emb_kernel(i0, i1, i2, i3, w0, w1, w2, w3, out,
                x0, x1, x2, x3, slot0, slot1,
                gat_sems, out_sem):
    wid = lax.axis_index("s") * 2 + lax.axis_index("c")
    idx_hbm = (i0, i1, i2, i3)
    tables = (w0, w1, w2, w3)
    idx_v = (x0, x1, x2, x3)
    slots = (slot0, slot1)

    for t in range(4):
        pltpu.sync_copy(idx_hbm[t].at[pl.ds(wid * _ROWS_PER_W, _ROWS_PER_W)],
                        idx_v[t])

    def fire_gathers(ci, j):
        for t in range(4):
            pltpu.async_copy(
                tables[t].at[idx_v[t].at[pl.ds(ci * _CHUNK, _CHUNK)]],
                slots[j][t], gat_sems[j])

    def wait_gathers(j):
        for t in range(4):
            pltpu.make_async_copy(tables[t].at[pl.ds(0, _CHUNK)],
                                  slots[j][t], gat_sems[j]).wait()

    for j in range(_DEPTH):
        fire_gathers(j, j)

    def body(k, carry):
        for j in range(_DEPTH):
            ci = k * _DEPTH + j
            base = wid * _ROWS_PER_W + ci * _CHUNK
            wait_gathers(j)
            writes = [
                pltpu.async_copy(
                    slots[j][t].at[:, pl.ds(0, _EMB[t])],
                    out.at[pl.ds(base, _CHUNK), pl.ds(_OFF[t], _EMB[t])],
                    out_sem)
                for t in range(4)]
            for w in writes:
                w.wait()

            @pl.when(k < _NOUTER - 1)
            def _():
                fire_gathers(ci + _DEPTH, j)

        return carry

    lax.fori_loop(0, _NOUTER, body, 0)


def kernel(inputs_0, inputs_1, inputs_2, inputs_3, W0, W1, W2, W3):
    idxs = [x.astype(jnp.int32).reshape(_N)
            for x in (inputs_0, inputs_1, inputs_2, inputs_3)]
    tabs = [jnp.pad(w, ((0, 0), (0, _PADW - w.shape[1])))
            for w in (W0, W1, W2, W3)]
    out = _emb_kernel(*idxs, *tabs)
    return out.reshape(_B, _L, _DTOT)


# speedup vs baseline: 1.1939x; 1.1939x over previous
import functools

import jax
import jax.numpy as jnp
from jax import lax
from jax.experimental import pallas as pl
from jax.experimental.pallas import tpu as pltpu
from jax.experimental.pallas import tpu_sc as plsc

_EMB = (64, 32, 32, 16)
_OFF = (0, 64, 96, 128)
_PADW = 128
_B, _L = 4096, 50
_N = _B * _L
_DTOT = 144
_NW = 32
_ROWS_PER_W = _N // _NW
_CHUNK = 80
_NCHUNK = _ROWS_PER_W // _CHUNK
_DEPTH = 2
_NOUTER = _NCHUNK // _DEPTH

_mesh = plsc.VectorSubcoreMesh(core_axis_name="c", subcore_axis_name="s")


def _stage_rows(wt, bv):
    e, v = wt.shape
    grid = (v + bv - 1) // bv

    def body(wt_ref, out_ref):
        out_ref[:, 0:e] = wt_ref[...].T

    return pl.pallas_call(
        body,
        grid=(grid,),
        in_specs=[pl.BlockSpec((e, bv), lambda i: (0, i))],
        out_specs=pl.BlockSpec((bv, _PADW), lambda i: (i, 0)),
        out_shape=jax.ShapeDtypeStruct((grid * bv, _PADW), jnp.float32),
    )(wt)


@functools.partial(
    pl.kernel,
    mesh=_mesh,
    out_type=jax.ShapeDtypeStruct((_N, _DTOT), jnp.float32),
    scratch_types=(
        [pltpu.VMEM((_ROWS_PER_W,), jnp.int32) for _ in range(4)]
        + [[pltpu.VMEM((_CHUNK, _PADW), jnp.float32) for _ in range(4)]
           for _ in range(_DEPTH)]
        + [[pltpu.SemaphoreType.DMA for _ in range(_DEPTH)],
           pltpu.SemaphoreType.DMA]
    ),
    compiler_params=pltpu.CompilerParams(use_tc_tiling_on_sc=False),
)
def _emb_kernel(i0, i1, i2, i3, w0, w1, w2, w3, out,
                x0, x1, x2, x3, slot0, slot1,
                gat_sems, out_sem):
    wid = lax.axis_index("s") * 2 + lax.axis_index("c")
    idx_hbm = (i0, i1, i2, i3)
    tables = (w0, w1, w2, w3)
    idx_v = (x0, x1, x2, x3)
    slots = (slot0, slot1)

    for t in range(4):
        pltpu.sync_copy(idx_hbm[t].at[pl.ds(wid * _ROWS_PER_W, _ROWS_PER_W)],
                        idx_v[t])

    def fire_gathers(ci, j):
        for t in range(4):
            pltpu.async_copy(
                tables[t].at[idx_v[t].at[pl.ds(ci * _CHUNK, _CHUNK)]],
                slots[j][t], gat_sems[j])

    def wait_gathers(j):
        for t in range(4):
            pltpu.make_async_copy(tables[t].at[pl.ds(0, _CHUNK)],
                                  slots[j][t], gat_sems[j]).wait()

    for j in range(_DEPTH):
        fire_gathers(j, j)

    def body(k, carry):
        for j in range(_DEPTH):
            ci = k * _DEPTH + j
            base = wid * _ROWS_PER_W + ci * _CHUNK
            wait_gathers(j)
            writes = [
                pltpu.async_copy(
                    slots[j][t].at[:, pl.ds(0, _EMB[t])],
                    out.at[pl.ds(base, _CHUNK), pl.ds(_OFF[t], _EMB[t])],
                    out_sem)
                for t in range(4)]
            for w in writes:
                w.wait()

            @pl.when(k < _NOUTER - 1)
            def _():
                fire_gathers(ci + _DEPTH, j)

        return carry

    lax.fori_loop(0, _NOUTER, body, 0)


def kernel(inputs_0, inputs_1, inputs_2, inputs_3, W0, W1, W2, W3):
    idxs = [x.astype(jnp.int32).reshape(_N)
            for x in (inputs_0, inputs_1, inputs_2, inputs_3)]
    tabs = [_stage_rows(w.T, 4096) for w in (W0, W1, W2)]
    tabs.append(jnp.pad(W3, ((0, 0), (0, _PADW - W3.shape[1]))))
    out = _emb_kernel(*idxs, *tabs)
    return out.reshape(_B, _L, _DTOT)


# speedup vs baseline: 1.3017x; 1.0903x over previous
import functools

import jax
import jax.numpy as jnp
from jax import lax
from jax.experimental import pallas as pl
from jax.experimental.pallas import tpu as pltpu
from jax.experimental.pallas import tpu_sc as plsc

_EMB = (64, 32, 32, 16)
_OFF = (0, 64, 96, 128)
_PADW = 128
_B, _L = 4096, 50
_N = _B * _L
_DTOT = 144
_NW = 32
_ROWS_PER_W = _N // _NW
_CHUNK = 80
_NCHUNK = _ROWS_PER_W // _CHUNK
_DEPTH = 2
_NOUTER = _NCHUNK // _DEPTH

_mesh = plsc.VectorSubcoreMesh(core_axis_name="c", subcore_axis_name="s")


def _stage_rows(wt, bv):
    e, v = wt.shape
    grid = (v + bv - 1) // bv

    def body(wt_ref, out_ref):
        out_ref[:, 0:e] = wt_ref[...].T

    return pl.pallas_call(
        body,
        grid=(grid,),
        in_specs=[pl.BlockSpec((e, bv), lambda i: (0, i))],
        out_specs=pl.BlockSpec((bv, _PADW), lambda i: (i, 0)),
        out_shape=jax.ShapeDtypeStruct((grid * bv, _PADW), jnp.float32),
    )(wt)


@functools.partial(
    pl.kernel,
    mesh=_mesh,
    out_type=jax.ShapeDtypeStruct((_N, _DTOT), jnp.float32),
    scratch_types=(
        [pltpu.VMEM((_ROWS_PER_W,), jnp.int32) for _ in range(4)]
        + [[pltpu.VMEM((_CHUNK, _PADW), jnp.float32) for _ in range(4)]
           for _ in range(_DEPTH)]
        + [[pltpu.SemaphoreType.DMA for _ in range(_DEPTH)],
           pltpu.SemaphoreType.DMA]
    ),
    compiler_params=pltpu.CompilerParams(use_tc_tiling_on_sc=False),
)
def _emb_kernel(i0, i1, i2, i3, w0, w1, w2, w3, out,
                x0, x1, x2, x3, slot0, slot1,
                gat_sems, out_sem):
    wid = lax.axis_index("s") * 2 + lax.axis_index("c")
    idx_hbm = (i0, i1, i2, i3)
    tables = (w0, w1, w2, w3)
    idx_v = (x0, x1, x2, x3)
    slots = (slot0, slot1)

    for t in range(4):
        pltpu.sync_copy(idx_hbm[t].at[pl.ds(wid * _ROWS_PER_W, _ROWS_PER_W)],
                        idx_v[t])

    def fire_gathers(ci, j):
        for t in range(4):
            pltpu.async_copy(
                tables[t].at[idx_v[t].at[pl.ds(ci * _CHUNK, _CHUNK)]],
                slots[j][t], gat_sems[j])

    def wait_gathers(j):
        for t in range(4):
            pltpu.make_async_copy(tables[t].at[pl.ds(0, _CHUNK)],
                                  slots[j][t], gat_sems[j]).wait()

    for j in range(_DEPTH):
        fire_gathers(j, j)

    def body(k, carry):
        for j in range(_DEPTH):
            ci = k * _DEPTH + j
            base = wid * _ROWS_PER_W + ci * _CHUNK
            wait_gathers(j)
            writes = [
                pltpu.async_copy(
                    slots[j][t].at[:, pl.ds(0, _EMB[t])],
                    out.at[pl.ds(base, _CHUNK), pl.ds(_OFF[t], _EMB[t])],
                    out_sem)
                for t in range(4)]
            for w in writes:
                w.wait()

            @pl.when(k < _NOUTER - 1)
            def _():
                fire_gathers(ci + _DEPTH, j)

        return carry

    lax.fori_loop(0, _NOUTER, body, 0)


def kernel(inputs_0, inputs_1, inputs_2, inputs_3, W0, W1, W2, W3):
    idxs = [x.astype(jnp.int32).reshape(_N)
            for x in (inputs_0, inputs_1, inputs_2, inputs_3)]
    tabs = [_stage_rows(W0.T, 8192), _stage_rows(W1.T, 4096),
            _stage_rows(W2.T, 4096)]
    tabs.append(jnp.pad(W3, ((0, 0), (0, _PADW - W3.shape[1]))))
    out = _emb_kernel(*idxs, *tabs)
    return out.reshape(_B, _L, _DTOT)


# speedup vs baseline: 1.3407x; 1.0300x over previous
import functools

import jax
import jax.numpy as jnp
from jax import lax
from jax.experimental import pallas as pl
from jax.experimental.pallas import tpu as pltpu
from jax.experimental.pallas import tpu_sc as plsc

_EMB = (64, 32, 32, 16)
_OFF = (0, 64, 96, 128)
_PADW = 128
_B, _L = 4096, 50
_N = _B * _L
_DTOT = 144
_NW = 32
_ROWS_PER_W = _N // _NW
_CHUNK = 80
_NCHUNK = _ROWS_PER_W // _CHUNK
_DEPTH = 2
_NOUTER = _NCHUNK // _DEPTH

_mesh = plsc.VectorSubcoreMesh(core_axis_name="c", subcore_axis_name="s")


def _stage_rows(wt, bv):
    e, v = wt.shape
    grid = (v + bv - 1) // bv

    def body(wt_ref, out_ref):
        out_ref[:, 0:e] = wt_ref[...].T

    return pl.pallas_call(
        body,
        grid=(grid,),
        in_specs=[pl.BlockSpec((e, bv), lambda i: (0, i))],
        out_specs=pl.BlockSpec((bv, _PADW), lambda i: (i, 0)),
        out_shape=jax.ShapeDtypeStruct((grid * bv, _PADW), jnp.float32),
    )(wt)


@functools.partial(
    pl.kernel,
    mesh=_mesh,
    out_type=jax.ShapeDtypeStruct((_N, _DTOT), jnp.float32),
    scratch_types=(
        [pltpu.VMEM((_ROWS_PER_W,), jnp.int32) for _ in range(4)]
        + [[pltpu.VMEM((_CHUNK, _PADW), jnp.float32) for _ in range(4)]
           for _ in range(_DEPTH)]
        + [[pltpu.SemaphoreType.DMA for _ in range(_DEPTH)],
           pltpu.SemaphoreType.DMA]
    ),
    compiler_params=pltpu.CompilerParams(use_tc_tiling_on_sc=False),
)
def _emb_kernel(i0, i1, i2, i3, w0, w1, w2, w3, out,
                x0, x1, x2, x3, slot0, slot1,
                gat_sems, out_sem):
    wid = lax.axis_index("s") * 2 + lax.axis_index("c")
    idx_hbm = (i0, i1, i2, i3)
    tables = (w0, w1, w2, w3)
    idx_v = (x0, x1, x2, x3)
    slots = (slot0, slot1)

    for t in range(4):
        pltpu.sync_copy(idx_hbm[t].at[pl.ds(wid * _ROWS_PER_W, _ROWS_PER_W)],
                        idx_v[t])

    def fire_gathers(ci, j):
        for t in range(4):
            pltpu.async_copy(
                tables[t].at[idx_v[t].at[pl.ds(ci * _CHUNK, _CHUNK)]],
                slots[j][t], gat_sems[j])

    def wait_gathers(j):
        for t in range(4):
            pltpu.make_async_copy(tables[t].at[pl.ds(0, _CHUNK)],
                                  slots[j][t], gat_sems[j]).wait()

    for j in range(_DEPTH):
        fire_gathers(j, j)

    def body(k, carry):
        for j in range(_DEPTH):
            ci = k * _DEPTH + j
            base = wid * _ROWS_PER_W + ci * _CHUNK
            wait_gathers(j)
            writes = [
                pltpu.async_copy(
                    slots[j][t].at[:, pl.ds(0, _EMB[t])],
                    out.at[pl.ds(base, _CHUNK), pl.ds(_OFF[t], _EMB[t])],
                    out_sem)
                for t in range(4)]
            for w in writes:
                w.wait()

            @pl.when(k < _NOUTER - 1)
            def _():
                fire_gathers(ci + _DEPTH, j)

        return carry

    lax.fori_loop(0, _NOUTER, body, 0)


def kernel(inputs_0, inputs_1, inputs_2, inputs_3, W0, W1, W2, W3):
    idxs = [x.astype(jnp.int32).reshape(_N)
            for x in (inputs_0, inputs_1, inputs_2, inputs_3)]
    tabs = [_stage_rows(W0.T, 16384), _stage_rows(W1.T, 8192),
            _stage_rows(W2.T, 8192)]
    tabs.append(jnp.pad(W3, ((0, 0), (0, _PADW - W3.shape[1]))))
    out = _emb_kernel(*idxs, *tabs)
    return out.reshape(_B, _L, _DTOT)


# speedup vs baseline: 1.3582x; 1.0130x over previous
import functools

import jax
import jax.numpy as jnp
from jax import lax
from jax.experimental import pallas as pl
from jax.experimental.pallas import tpu as pltpu
from jax.experimental.pallas import tpu_sc as plsc

_EMB = (64, 32, 32, 16)
_OFF = (0, 64, 96, 128)
_PADW = 128
_B, _L = 4096, 50
_N = _B * _L
_DTOT = 144
_NW = 32
_ROWS_PER_W = _N // _NW
_CHUNK = 80
_NCHUNK = _ROWS_PER_W // _CHUNK
_DEPTH = 2
_NOUTER = _NCHUNK // _DEPTH

_mesh = plsc.VectorSubcoreMesh(core_axis_name="c", subcore_axis_name="s")


def _stage_rows(wt, bv):
    e, v = wt.shape
    grid = (v + bv - 1) // bv

    def body(wt_ref, out_ref):
        out_ref[:, 0:e] = wt_ref[...].T

    return pl.pallas_call(
        body,
        grid=(grid,),
        in_specs=[pl.BlockSpec((e, bv), lambda i: (0, i))],
        out_specs=pl.BlockSpec((bv, _PADW), lambda i: (i, 0)),
        out_shape=jax.ShapeDtypeStruct((grid * bv, _PADW), jnp.float32),
    )(wt)


@functools.partial(
    pl.kernel,
    mesh=_mesh,
    out_type=jax.ShapeDtypeStruct((_N, _DTOT), jnp.float32),
    scratch_types=(
        [pltpu.VMEM((_ROWS_PER_W,), jnp.int32) for _ in range(4)]
        + [[pltpu.VMEM((_CHUNK, _PADW), jnp.float32) for _ in range(4)]
           for _ in range(_DEPTH)]
        + [[pltpu.SemaphoreType.DMA for _ in range(_DEPTH)],
           pltpu.SemaphoreType.DMA]
    ),
    compiler_params=pltpu.CompilerParams(use_tc_tiling_on_sc=False),
)
def _emb_kernel(i0, i1, i2, i3, w0, w1, w2, w3, out,
                x0, x1, x2, x3, slot0, slot1,
                gat_sems, out_sem):
    wid = lax.axis_index("s") * 2 + lax.axis_index("c")
    idx_hbm = (i0, i1, i2, i3)
    tables = (w0, w1, w2, w3)
    idx_v = (x0, x1, x2, x3)
    slots = (slot0, slot1)

    for t in range(4):
        pltpu.sync_copy(idx_hbm[t].at[pl.ds(wid * _ROWS_PER_W, _ROWS_PER_W)],
                        idx_v[t])

    def fire_gathers(ci, j):
        for t in range(4):
            pltpu.async_copy(
                tables[t].at[idx_v[t].at[pl.ds(ci * _CHUNK, _CHUNK)]],
                slots[j][t], gat_sems[j])

    def wait_gathers(j):
        for t in range(4):
            pltpu.make_async_copy(tables[t].at[pl.ds(0, _CHUNK)],
                                  slots[j][t], gat_sems[j]).wait()

    for j in range(_DEPTH):
        fire_gathers(j, j)

    def body(k, carry):
        for j in range(_DEPTH):
            ci = k * _DEPTH + j
            base = wid * _ROWS_PER_W + ci * _CHUNK
            wait_gathers(j)
            writes = [
                pltpu.async_copy(
                    slots[j][t].at[:, pl.ds(0, _EMB[t])],
                    out.at[pl.ds(base, _CHUNK), pl.ds(_OFF[t], _EMB[t])],
                    out_sem)
                for t in range(4)]
            for w in writes:
                w.wait()

            @pl.when(k < _NOUTER - 1)
            def _():
                fire_gathers(ci + _DEPTH, j)

        return carry

    lax.fori_loop(0, _NOUTER, body, 0)


def kernel(inputs_0, inputs_1, inputs_2, inputs_3, W0, W1, W2, W3):
    idxs = [x.astype(jnp.int32).reshape(_N)
            for x in (inputs_0, inputs_1, inputs_2, inputs_3)]
    tabs = [_stage_rows(W0.T, 32768), _stage_rows(W1.T, 16384),
            _stage_rows(W2.T, 16384)]
    tabs.append(jnp.pad(W3, ((0, 0), (0, _PADW - W3.shape[1]))))
    out = _emb_kernel(*idxs, *tabs)
    return out.reshape(_B, _L, _DTOT)


# speedup vs baseline: 1.3626x; 1.0033x over previous
import functools

import jax
import jax.numpy as jnp
from jax import lax
from jax.experimental import pallas as pl
from jax.experimental.pallas import tpu as pltpu
from jax.experimental.pallas import tpu_sc as plsc

_EMB = (64, 32, 32, 16)
_OFF = (0, 64, 96, 128)
_PADW = 128
_B, _L = 4096, 50
_N = _B * _L
_DTOT = 144
_NW = 32
_ROWS_PER_W = _N // _NW
_CHUNK = 40
_NCHUNK = _ROWS_PER_W // _CHUNK
_DEPTH = 4
_NOUTER = _NCHUNK // _DEPTH

_mesh = plsc.VectorSubcoreMesh(core_axis_name="c", subcore_axis_name="s")


def _stage_rows(wt, bv):
    e, v = wt.shape
    grid = (v + bv - 1) // bv

    def body(wt_ref, out_ref):
        out_ref[:, 0:e] = wt_ref[...].T

    return pl.pallas_call(
        body,
        grid=(grid,),
        in_specs=[pl.BlockSpec((e, bv), lambda i: (0, i))],
        out_specs=pl.BlockSpec((bv, _PADW), lambda i: (i, 0)),
        out_shape=jax.ShapeDtypeStruct((grid * bv, _PADW), jnp.float32),
    )(wt)


@functools.partial(
    pl.kernel,
    mesh=_mesh,
    out_type=jax.ShapeDtypeStruct((_N, _DTOT), jnp.float32),
    scratch_types=(
        [pltpu.VMEM((_ROWS_PER_W,), jnp.int32) for _ in range(4)]
        + [[pltpu.VMEM((_CHUNK, _PADW), jnp.float32) for _ in range(4)]
           for _ in range(_DEPTH)]
        + [[pltpu.SemaphoreType.DMA for _ in range(_DEPTH)],
           pltpu.SemaphoreType.DMA]
    ),
    compiler_params=pltpu.CompilerParams(use_tc_tiling_on_sc=False),
)
def _emb_kernel(i0, i1, i2, i3, w0, w1, w2, w3, out,
                x0, x1, x2, x3, slot0, slot1, slot2, slot3,
                gat_sems, out_sem):
    wid = lax.axis_index("s") * 2 + lax.axis_index("c")
    idx_hbm = (i0, i1, i2, i3)
    tables = (w0, w1, w2, w3)
    idx_v = (x0, x1, x2, x3)
    slots = (slot0, slot1, slot2, slot3)

    for t in range(4):
        pltpu.sync_copy(idx_hbm[t].at[pl.ds(wid * _ROWS_PER_W, _ROWS_PER_W)],
                        idx_v[t])

    def fire_gathers(ci, j):
        for t in range(4):
            pltpu.async_copy(
                tables[t].at[idx_v[t].at[pl.ds(ci * _CHUNK, _CHUNK)]],
                slots[j][t], gat_sems[j])

    def wait_gathers(j):
        for t in range(4):
            pltpu.make_async_copy(tables[t].at[pl.ds(0, _CHUNK)],
                                  slots[j][t], gat_sems[j]).wait()

    for j in range(_DEPTH):
        fire_gathers(j, j)

    def body(k, carry):
        for j in range(_DEPTH):
            ci = k * _DEPTH + j
            base = wid * _ROWS_PER_W + ci * _CHUNK
            wait_gathers(j)
            writes = [
                pltpu.async_copy(
                    slots[j][t].at[:, pl.ds(0, _EMB[t])],
                    out.at[pl.ds(base, _CHUNK), pl.ds(_OFF[t], _EMB[t])],
                    out_sem)
                for t in range(4)]
            for w in writes:
                w.wait()

            @pl.when(k < _NOUTER - 1)
            def _():
                fire_gathers(ci + _DEPTH, j)

        return carry

    lax.fori_loop(0, _NOUTER, body, 0)


def kernel(inputs_0, inputs_1, inputs_2, inputs_3, W0, W1, W2, W3):
    idxs = [x.astype(jnp.int32).reshape(_N)
            for x in (inputs_0, inputs_1, inputs_2, inputs_3)]
    tabs = [_stage_rows(W0.T, 32768), _stage_rows(W1.T, 16384),
            _stage_rows(W2.T, 16384)]
    tabs.append(jnp.pad(W3, ((0, 0), (0, _PADW - W3.shape[1]))))
    out = _emb_kernel(*idxs, *tabs)
    return out.reshape(_B, _L, _DTOT)
